# Initial kernel scaffold; baseline (speedup 1.0000x reference)
import jax
import jax.numpy as jnp
from jax.experimental import pallas as pl


def kernel(x, edge_index, batch, item_embedding, W1, b1, W2, b2, W3, b3):
    raise NotImplementedError("write your pallas kernel here")


# speedup vs baseline: 13.4704x; 13.4704x over previous
import jax
import jax.numpy as jnp
from jax import lax
from jax.experimental import pallas as pl
from jax.experimental.pallas import tpu as pltpu
from jax.experimental.pallas import tpu_sc as plsc

_N = 10000
_VOCAB = 100
_EMBED = 10
_H = 128
_E = 320000

_NW = 32
_NPAD = 10240
_NPW = _NPAD // _NW
_GCH = 80
_CHUNK = 128
_EPW = 10112
_EPAD = _NW * _EPW
_NCHUNK = _EPW // _CHUNK
_TPW = _NPAD // 16
_DUMMY = _N


def _mesh():
    return plsc.VectorSubcoreMesh(core_axis_name="c", subcore_axis_name="s")


def _sc_deg_emb(xp, dst, table):
    def body(x_hbm, dst_hbm, table_hbm, deg_out, emb_out,
             acc, xidx, grows, didx, ones, zbuf, sem):
        c = lax.axis_index("c")
        s = lax.axis_index("s")
        wid = s * 2 + c

        def fill_ones(i, carry):
            ones[i] = jnp.ones((16,), jnp.float32)
            return carry

        lax.fori_loop(0, _CHUNK, fill_ones, 0)

        def fill_zero(i, carry):
            zbuf[i] = jnp.zeros((16,), jnp.float32)
            return carry

        lax.fori_loop(0, _TPW, fill_zero, 0)
        pltpu.sync_copy(zbuf, acc.at[pl.ds(s * _TPW, _TPW)])

        for j in range(_NPW // _GCH):
            b = wid * _NPW + j * _GCH
            pltpu.sync_copy(x_hbm.at[pl.ds(b, _GCH)], xidx)
            pltpu.async_copy(table_hbm.at[xidx], grows, sem).wait()
            pltpu.sync_copy(grows, emb_out.at[pl.ds(b, _GCH)])

        plsc.subcore_barrier()

        def deg_step(t, carry):
            base = wid * _EPW + t * _CHUNK
            pltpu.sync_copy(dst_hbm.at[pl.ds(base, _CHUNK)], didx)
            pltpu.sync_copy(ones, acc.at[didx], add=True)
            return carry

        lax.fori_loop(0, _NCHUNK, deg_step, 0)
        plsc.subcore_barrier()
        pltpu.sync_copy(acc.at[pl.ds(s * _TPW, _TPW)],
                        deg_out.at[pl.ds(c * _NPAD + s * _TPW, _TPW)])

    f = pl.kernel(
        body,
        out_type=[jax.ShapeDtypeStruct((2 * _NPAD, 16), jnp.float32),
                  jax.ShapeDtypeStruct((_NPAD, 16), jnp.float32)],
        mesh=_mesh(),
        compiler_params=pltpu.CompilerParams(use_tc_tiling_on_sc=False),
        scratch_types=[
            pltpu.VMEM_SHARED((_NPAD, 16), jnp.float32),
            pltpu.VMEM((_GCH,), jnp.int32),
            pltpu.VMEM((_GCH, 16), jnp.float32),
            pltpu.VMEM((_CHUNK,), jnp.int32),
            pltpu.VMEM((_CHUNK, 16), jnp.float32),
            pltpu.VMEM((_TPW, 16), jnp.float32),
            pltpu.SemaphoreType.DMA,
        ],
    )
    return f(xp, dst, table)


def _sc_scatter(src, dst, g, D, zrows):
    nz = _TPW // zrows

    def body(src_hbm, dst_hbm, g_hbm, out, acc, sidx, didx, rows, zbuf, sem):
        c = lax.axis_index("c")
        s = lax.axis_index("s")
        wid = s * 2 + c

        def fill_zero(i, carry):
            for j in range(D // 16):
                zbuf[i, pl.ds(j * 16, 16)] = jnp.zeros((16,), jnp.float32)
            return carry

        lax.fori_loop(0, zrows, fill_zero, 0)
        for k in range(nz):
            pltpu.sync_copy(zbuf, acc.at[pl.ds(s * _TPW + k * zrows, zrows)])
        plsc.subcore_barrier()

        def step(t, carry):
            base = wid * _EPW + t * _CHUNK
            pltpu.sync_copy(src_hbm.at[pl.ds(base, _CHUNK)], sidx)
            pltpu.async_copy(g_hbm.at[sidx], rows, sem).wait()
            pltpu.sync_copy(dst_hbm.at[pl.ds(base, _CHUNK)], didx)
            pltpu.sync_copy(rows, acc.at[didx], add=True)
            return carry

        lax.fori_loop(0, _NCHUNK, step, 0)
        plsc.subcore_barrier()
        pltpu.sync_copy(acc.at[pl.ds(s * _TPW, _TPW)],
                        out.at[pl.ds(c * _NPAD + s * _TPW, _TPW)])

    f = pl.kernel(
        body,
        out_type=jax.ShapeDtypeStruct((2 * _NPAD, D), jnp.float32),
        mesh=_mesh(),
        compiler_params=pltpu.CompilerParams(use_tc_tiling_on_sc=False),
        scratch_types=[
            pltpu.VMEM_SHARED((_NPAD, D), jnp.float32),
            pltpu.VMEM((_CHUNK,), jnp.int32),
            pltpu.VMEM((_CHUNK,), jnp.int32),
            pltpu.VMEM((_CHUNK, D), jnp.float32),
            pltpu.VMEM((zrows, D), jnp.float32),
            pltpu.SemaphoreType.DMA,
        ],
    )
    return f(src, dst, g)


_BLK = 1280


def _tc_a(degp, emb):
    def body(dp, em, g1_ref):
        d = dp[0] + dp[1] + 1.0
        g1_ref[...] = lax.rsqrt(d) * em[...]

    return pl.pallas_call(
        body,
        grid=(_NPAD // _BLK,),
        in_specs=[pl.BlockSpec((2, _BLK, 16), lambda i: (0, i, 0)),
                  pl.BlockSpec((_BLK, 16), lambda i: (i, 0))],
        out_specs=pl.BlockSpec((_BLK, 16), lambda i: (i, 0)),
        out_shape=jax.ShapeDtypeStruct((_NPAD, 16), jnp.float32),
    )(degp, emb)


def _tc_b(degp, scat1p, g1, W1p, b1):
    def body(dp, s1, g1r, w, b, g2_ref):
        dinv16 = lax.rsqrt(dp[0] + dp[1] + 1.0)
        agg = dinv16 * (s1[0] + s1[1] + g1r[...])
        h1 = jnp.maximum(
            jnp.dot(agg, w[...], preferred_element_type=jnp.float32) + b[...],
            0.0)
        g2_ref[...] = dinv16[:, :1] * h1

    return pl.pallas_call(
        body,
        grid=(_NPAD // _BLK,),
        in_specs=[pl.BlockSpec((2, _BLK, 16), lambda i: (0, i, 0)),
                  pl.BlockSpec((2, _BLK, 16), lambda i: (0, i, 0)),
                  pl.BlockSpec((_BLK, 16), lambda i: (i, 0)),
                  pl.BlockSpec((16, _H), lambda i: (0, 0)),
                  pl.BlockSpec((1, _H), lambda i: (0, 0))],
        out_specs=pl.BlockSpec((_BLK, _H), lambda i: (i, 0)),
        out_shape=jax.ShapeDtypeStruct((_NPAD, _H), jnp.float32),
    )(degp, scat1p, g1, W1p, b1)


def _tc_c(degp, scat2p, g2, W2, b2, W3r, b3):
    def body(dp, s2, g2r, w2, b2r, w3, b3r, out_ref):
        dinv = lax.rsqrt(dp[0, :, :1] + dp[1, :, :1] + 1.0)
        agg = dinv * (s2[0] + s2[1] + g2r[...])
        h2 = jnp.maximum(
            jnp.dot(agg, w2[...], preferred_element_type=jnp.float32)
            + b2r[...], 0.0)
        z = jnp.sum(h2 * w3[...], axis=1, keepdims=True) + b3r[...]
        out_ref[...] = jax.nn.sigmoid(z)

    return pl.pallas_call(
        body,
        grid=(_NPAD // _BLK,),
        in_specs=[pl.BlockSpec((2, _BLK, 16), lambda i: (0, i, 0)),
                  pl.BlockSpec((2, _BLK, _H), lambda i: (0, i, 0)),
                  pl.BlockSpec((_BLK, _H), lambda i: (i, 0)),
                  pl.BlockSpec((_H, _H), lambda i: (0, 0)),
                  pl.BlockSpec((1, _H), lambda i: (0, 0)),
                  pl.BlockSpec((1, _H), lambda i: (0, 0)),
                  pl.BlockSpec((1, 1), lambda i: (0, 0))],
        out_specs=pl.BlockSpec((_BLK, 1), lambda i: (i, 0)),
        out_shape=jax.ShapeDtypeStruct((_NPAD, 1), jnp.float32),
    )(degp, scat2p, g2, W2, b2, W3r, b3)


def kernel(x, edge_index, batch, item_embedding, W1, b1, W2, b2, W3, b3):
    f32 = jnp.float32
    xp = jnp.zeros((_NPAD,), jnp.int32).at[:_N].set(x[:, 0])
    src = jnp.full((_EPAD,), _DUMMY, jnp.int32).at[:_E].set(edge_index[0])
    dst = jnp.full((_EPAD,), _DUMMY, jnp.int32).at[:_E].set(edge_index[1])
    table = jnp.zeros((_VOCAB, 16), f32).at[:, :_EMBED].set(item_embedding)
    W1p = jnp.zeros((16, _H), f32).at[:_EMBED].set(W1)

    degf, emb = _sc_deg_emb(xp, dst, table)
    degp = degf.reshape(2, _NPAD, 16)
    g1 = _tc_a(degp, emb)
    scat1p = _sc_scatter(src, dst, g1, 16, _TPW).reshape(2, _NPAD, 16)
    g2 = _tc_b(degp, scat1p, g1, W1p, b1.reshape(1, _H))
    scat2p = _sc_scatter(src, dst, g2, _H, 64).reshape(2, _NPAD, _H)
    out = _tc_c(degp, scat2p, g2, W2, b2.reshape(1, _H),
                W3.reshape(1, _H), b3.reshape(1, 1))
    return out[:_N, 0]


# speedup vs baseline: 32.3768x; 2.4036x over previous
import jax
import jax.numpy as jnp
from jax import lax
from jax.experimental import pallas as pl
from jax.experimental.pallas import tpu as pltpu
from jax.experimental.pallas import tpu_sc as plsc

_N = 10000
_VOCAB = 100
_EMBED = 10
_H = 128
_E = 320000

_NW = 32
_NPAD = 10240
_NPW = _NPAD // _NW
_GCH = 80
_CHUNK = 128
_NCHUNK = 80
_EPW = _NCHUNK * _CHUNK
_EPAD = _NW * _EPW
_NBUF = 4
_TPW = _NPAD // 16


def _mesh():
    return plsc.VectorSubcoreMesh(core_axis_name="c", subcore_axis_name="s")


def _sc_deg_emb(xp, dst2d, table):
    def body(x_hbm, dst_hbm, table_hbm, deg_out, emb_out,
             acc, xidx, grows, didx_all, ones, zbuf, sem):
        c = lax.axis_index("c")
        s = lax.axis_index("s")
        wid = s * 2 + c

        pltpu.sync_copy(dst_hbm.at[pl.ds(wid * _NCHUNK, _NCHUNK)], didx_all)

        def fill_ones(i, carry):
            ones[i] = jnp.ones((16,), jnp.float32)
            return carry

        lax.fori_loop(0, _CHUNK, fill_ones, 0)

        def fill_zero(i, carry):
            zbuf[i] = jnp.zeros((16,), jnp.float32)
            return carry

        lax.fori_loop(0, _TPW, fill_zero, 0)
        pltpu.sync_copy(zbuf, acc.at[pl.ds(s * _TPW, _TPW)])

        for j in range(_NPW // _GCH):
            b = wid * _NPW + j * _GCH
            pltpu.sync_copy(x_hbm.at[pl.ds(b, _GCH)], xidx)
            pltpu.async_copy(table_hbm.at[xidx], grows, sem).wait()
            pltpu.sync_copy(grows, emb_out.at[pl.ds(b, _GCH)])

        plsc.subcore_barrier()

        def deg_step(t, carry):
            pltpu.sync_copy(ones, acc.at[didx_all.at[t]], add=True)
            return carry

        lax.fori_loop(0, _NCHUNK, deg_step, 0)
        plsc.subcore_barrier()
        pltpu.sync_copy(acc.at[pl.ds(s * _TPW, _TPW)],
                        deg_out.at[pl.ds(c * _NPAD + s * _TPW, _TPW)])

    f = pl.kernel(
        body,
        out_type=[jax.ShapeDtypeStruct((2 * _NPAD, 16), jnp.float32),
                  jax.ShapeDtypeStruct((_NPAD, 16), jnp.float32)],
        mesh=_mesh(),
        compiler_params=pltpu.CompilerParams(use_tc_tiling_on_sc=False),
        scratch_types=[
            pltpu.VMEM_SHARED((_NPAD, 16), jnp.float32),
            pltpu.VMEM((_GCH,), jnp.int32),
            pltpu.VMEM((_GCH, 16), jnp.float32),
            pltpu.VMEM((_NCHUNK, _CHUNK), jnp.int32),
            pltpu.VMEM((_CHUNK, 16), jnp.float32),
            pltpu.VMEM((_TPW, 16), jnp.float32),
            pltpu.SemaphoreType.DMA,
        ],
    )
    return f(xp, dst2d, table)


def _sc_scatter(src, dst, g, D, chunk, nbuf, zrows):
    nz = _TPW // zrows
    nchk = _EPW // chunk

    def body(src_hbm, dst_hbm, g_hbm, out,
             acc, sidx_all, didx_all, rows, zbuf, *sems):
        c = lax.axis_index("c")
        s = lax.axis_index("s")
        wid = s * 2 + c

        pltpu.sync_copy(src_hbm.at[pl.ds(wid * nchk, nchk)], sidx_all)
        pltpu.sync_copy(dst_hbm.at[pl.ds(wid * nchk, nchk)], didx_all)

        def fill_zero(i, carry):
            for j in range(D // 16):
                zbuf[i, pl.ds(j * 16, 16)] = jnp.zeros((16,), jnp.float32)
            return carry

        lax.fori_loop(0, zrows, fill_zero, 0)
        for k in range(nz):
            pltpu.sync_copy(zbuf, acc.at[pl.ds(s * _TPW + k * zrows, zrows)])
        plsc.subcore_barrier()

        def gather_start(t, b):
            pltpu.make_async_copy(
                g_hbm.at[sidx_all.at[t]], rows.at[b], sems[b]).start()

        def gather_wait(t, b):
            pltpu.make_async_copy(
                g_hbm.at[sidx_all.at[t]], rows.at[b], sems[b]).wait()

        for b in range(nbuf):
            gather_start(b, b)

        def step(grp, carry):
            for b in range(nbuf):
                t = nbuf * grp + b
                gather_wait(t, b)
                pltpu.sync_copy(rows.at[b], acc.at[didx_all.at[t]], add=True)
                gather_start(t + nbuf, b)
            return carry

        lax.fori_loop(0, nchk // nbuf - 1, step, 0)
        for b in range(nbuf):
            t = nchk - nbuf + b
            gather_wait(t, b)
            pltpu.sync_copy(rows.at[b], acc.at[didx_all.at[t]], add=True)

        plsc.subcore_barrier()
        pltpu.sync_copy(acc.at[pl.ds(s * _TPW, _TPW)],
                        out.at[pl.ds(c * _NPAD + s * _TPW, _TPW)])

    f = pl.kernel(
        body,
        out_type=jax.ShapeDtypeStruct((2 * _NPAD, D), jnp.float32),
        mesh=_mesh(),
        compiler_params=pltpu.CompilerParams(use_tc_tiling_on_sc=False),
        scratch_types=[
            pltpu.VMEM_SHARED((_NPAD, D), jnp.float32),
            pltpu.VMEM((nchk, chunk), jnp.int32),
            pltpu.VMEM((nchk, chunk), jnp.int32),
            pltpu.VMEM((nbuf, chunk, D), jnp.float32),
            pltpu.VMEM((zrows, D), jnp.float32),
        ] + [pltpu.SemaphoreType.DMA] * nbuf,
    )
    return f(src.reshape(-1, chunk), dst.reshape(-1, chunk), g)


_BLK = 1280


def _tc_a(degp, emb):
    def body(dp, em, g1_ref):
        d = dp[0] + dp[1] + 1.0
        g1_ref[...] = lax.rsqrt(d) * em[...]

    return pl.pallas_call(
        body,
        grid=(_NPAD // _BLK,),
        in_specs=[pl.BlockSpec((2, _BLK, 16), lambda i: (0, i, 0)),
                  pl.BlockSpec((_BLK, 16), lambda i: (i, 0))],
        out_specs=pl.BlockSpec((_BLK, 16), lambda i: (i, 0)),
        out_shape=jax.ShapeDtypeStruct((_NPAD, 16), jnp.float32),
    )(degp, emb)


def _tc_b(degp, scat1p, g1, W1p, b1):
    def body(dp, s1, g1r, w, b, g2_ref):
        dinv16 = lax.rsqrt(dp[0] + dp[1] + 1.0)
        agg = dinv16 * (s1[0] + s1[1] + g1r[...])
        h1 = jnp.maximum(
            jnp.dot(agg, w[...], preferred_element_type=jnp.float32) + b[...],
            0.0)
        g2_ref[...] = dinv16[:, :1] * h1

    return pl.pallas_call(
        body,
        grid=(_NPAD // _BLK,),
        in_specs=[pl.BlockSpec((2, _BLK, 16), lambda i: (0, i, 0)),
                  pl.BlockSpec((2, _BLK, 16), lambda i: (0, i, 0)),
                  pl.BlockSpec((_BLK, 16), lambda i: (i, 0)),
                  pl.BlockSpec((16, _H), lambda i: (0, 0)),
                  pl.BlockSpec((1, _H), lambda i: (0, 0))],
        out_specs=pl.BlockSpec((_BLK, _H), lambda i: (i, 0)),
        out_shape=jax.ShapeDtypeStruct((_NPAD, _H), jnp.float32),
    )(degp, scat1p, g1, W1p, b1)


def _tc_c(degp, scat2p, g2, W2, b2, W3r, b3):
    def body(dp, s2, g2r, w2, b2r, w3, b3r, out_ref):
        dinv = lax.rsqrt(dp[0, :, :1] + dp[1, :, :1] + 1.0)
        agg = dinv * (s2[0] + s2[1] + g2r[...])
        h2 = jnp.maximum(
            jnp.dot(agg, w2[...], preferred_element_type=jnp.float32)
            + b2r[...], 0.0)
        z = jnp.sum(h2 * w3[...], axis=1, keepdims=True) + b3r[...]
        out_ref[...] = jax.nn.sigmoid(z)

    return pl.pallas_call(
        body,
        grid=(_NPAD // _BLK,),
        in_specs=[pl.BlockSpec((2, _BLK, 16), lambda i: (0, i, 0)),
                  pl.BlockSpec((2, _BLK, _H), lambda i: (0, i, 0)),
                  pl.BlockSpec((_BLK, _H), lambda i: (i, 0)),
                  pl.BlockSpec((_H, _H), lambda i: (0, 0)),
                  pl.BlockSpec((1, _H), lambda i: (0, 0)),
                  pl.BlockSpec((1, _H), lambda i: (0, 0)),
                  pl.BlockSpec((1, 1), lambda i: (0, 0))],
        out_specs=pl.BlockSpec((_BLK, 1), lambda i: (i, 0)),
        out_shape=jax.ShapeDtypeStruct((_NPAD, 1), jnp.float32),
    )(degp, scat2p, g2, W2, b2, W3r, b3)


def kernel(x, edge_index, batch, item_embedding, W1, b1, W2, b2, W3, b3):
    f32 = jnp.float32
    xp = jnp.zeros((_NPAD,), jnp.int32).at[:_N].set(x[:, 0])
    pad_idx = _N + (jnp.arange(_EPAD - _E, dtype=jnp.int32) % (_NPAD - _N))
    src = jnp.concatenate([edge_index[0], pad_idx])
    dst = jnp.concatenate([edge_index[1], pad_idx])
    table = jnp.zeros((_VOCAB, 16), f32).at[:, :_EMBED].set(item_embedding)
    W1p = jnp.zeros((16, _H), f32).at[:_EMBED].set(W1)

    degf, emb = _sc_deg_emb(xp, dst.reshape(-1, _CHUNK), table)
    degp = degf.reshape(2, _NPAD, 16)
    g1 = _tc_a(degp, emb)
    scat1p = _sc_scatter(src, dst, g1, 16, 128, 4, _TPW).reshape(2, _NPAD, 16)
    g2 = _tc_b(degp, scat1p, g1, W1p, b1.reshape(1, _H))
    scat2p = _sc_scatter(src, dst, g2, _H, 64, 2, 16).reshape(2, _NPAD, _H)
    out = _tc_c(degp, scat2p, g2, W2, b2.reshape(1, _H),
                W3.reshape(1, _H), b3.reshape(1, 1))
    return out[:_N, 0]


# speedup vs baseline: 34.1321x; 1.0542x over previous
import jax
import jax.numpy as jnp
from jax import lax
from jax.experimental import pallas as pl
from jax.experimental.pallas import tpu as pltpu
from jax.experimental.pallas import tpu_sc as plsc

_N = 10000
_VOCAB = 100
_EMBED = 10
_H = 128
_E = 320000

_NW = 32
_NPAD = 10240
_NPW = _NPAD // _NW
_CH = 80
_NCHK = _E // (_NW * _CH)
_TPW = _NPAD // 16


def _mesh():
    return plsc.VectorSubcoreMesh(core_axis_name="c", subcore_axis_name="s")


def _ring(nbuf, gather_start, gather_wait, scatter):
    for b in range(nbuf):
        gather_start(b, b)
    fg = (_NCHK - nbuf) // nbuf

    def step(grp, carry):
        for b in range(nbuf):
            t = nbuf * grp + b
            gather_wait(t, b)
            scatter(t, b)
            gather_start(t + nbuf, b)
        return carry

    lax.fori_loop(0, fg, step, 0)
    for t in range(fg * nbuf, _NCHK):
        b = t % nbuf
        gather_wait(t, b)
        scatter(t, b)
        if t + nbuf < _NCHK:
            gather_start(t + nbuf, b)


def _sc_deg_emb(xp, ei3, table):
    def body(x_hbm, ei_hbm, table_hbm, deg_out, emb_out,
             acc, xidx, grows, didx_all, ones, zbuf, sem):
        c = lax.axis_index("c")
        s = lax.axis_index("s")
        wid = s * 2 + c

        pltpu.sync_copy(ei_hbm.at[1, pl.ds(wid * _NCHK, _NCHK)], didx_all)

        def fill_ones(i, carry):
            ones[i] = jnp.ones((16,), jnp.float32)
            return carry

        lax.fori_loop(0, _CH, fill_ones, 0)

        def fill_zero(i, carry):
            zbuf[i] = jnp.zeros((16,), jnp.float32)
            return carry

        lax.fori_loop(0, _TPW, fill_zero, 0)
        pltpu.sync_copy(zbuf, acc.at[pl.ds(s * _TPW, _TPW)])

        for j in range(_NPW // _CH):
            b = wid * _NPW + j * _CH
            pltpu.sync_copy(x_hbm.at[pl.ds(b, _CH)], xidx)
            pltpu.async_copy(table_hbm.at[xidx], grows, sem).wait()
            pltpu.sync_copy(grows, emb_out.at[pl.ds(b, _CH)])

        plsc.subcore_barrier()

        def deg_step(t, carry):
            pltpu.sync_copy(ones, acc.at[didx_all.at[t]], add=True)
            return carry

        lax.fori_loop(0, _NCHK, deg_step, 0)
        plsc.subcore_barrier()
        pltpu.sync_copy(acc.at[pl.ds(s * _TPW, _TPW)],
                        deg_out.at[pl.ds(c * _NPAD + s * _TPW, _TPW)])

    f = pl.kernel(
        body,
        out_type=[jax.ShapeDtypeStruct((2 * _NPAD, 16), jnp.float32),
                  jax.ShapeDtypeStruct((_NPAD, 16), jnp.float32)],
        mesh=_mesh(),
        compiler_params=pltpu.CompilerParams(use_tc_tiling_on_sc=False),
        scratch_types=[
            pltpu.VMEM_SHARED((_NPAD, 16), jnp.float32),
            pltpu.VMEM((_CH,), jnp.int32),
            pltpu.VMEM((_CH, 16), jnp.float32),
            pltpu.VMEM((_NCHK, _CH), jnp.int32),
            pltpu.VMEM((_CH, 16), jnp.float32),
            pltpu.VMEM((_TPW, 16), jnp.float32),
            pltpu.SemaphoreType.DMA,
        ],
    )
    return f(xp, ei3, table)


def _sc_scatter(ei3, g, D, nbuf, zrows):
    nz = _TPW // zrows

    def body(ei_hbm, g_hbm, out, acc, sidx_all, didx_all, rows, zbuf, *sems):
        c = lax.axis_index("c")
        s = lax.axis_index("s")
        wid = s * 2 + c

        pltpu.sync_copy(ei_hbm.at[0, pl.ds(wid * _NCHK, _NCHK)], sidx_all)
        pltpu.sync_copy(ei_hbm.at[1, pl.ds(wid * _NCHK, _NCHK)], didx_all)

        def fill_zero(i, carry):
            for j in range(D // 16):
                zbuf[i, pl.ds(j * 16, 16)] = jnp.zeros((16,), jnp.float32)
            return carry

        lax.fori_loop(0, zrows, fill_zero, 0)
        for k in range(nz):
            pltpu.sync_copy(zbuf, acc.at[pl.ds(s * _TPW + k * zrows, zrows)])
        plsc.subcore_barrier()

        def gather_start(t, b):
            pltpu.make_async_copy(
                g_hbm.at[sidx_all.at[t]], rows.at[b], sems[b]).start()

        def gather_wait(t, b):
            pltpu.make_async_copy(
                g_hbm.at[sidx_all.at[t]], rows.at[b], sems[b]).wait()

        def scatter(t, b):
            pltpu.sync_copy(rows.at[b], acc.at[didx_all.at[t]], add=True)

        _ring(nbuf, gather_start, gather_wait, scatter)

        plsc.subcore_barrier()
        pltpu.sync_copy(acc.at[pl.ds(s * _TPW, _TPW)],
                        out.at[pl.ds(c * _NPAD + s * _TPW, _TPW)])

    f = pl.kernel(
        body,
        out_type=jax.ShapeDtypeStruct((2 * _NPAD, D), jnp.float32),
        mesh=_mesh(),
        compiler_params=pltpu.CompilerParams(use_tc_tiling_on_sc=False),
        scratch_types=[
            pltpu.VMEM_SHARED((_NPAD, D), jnp.float32),
            pltpu.VMEM((_NCHK, _CH), jnp.int32),
            pltpu.VMEM((_NCHK, _CH), jnp.int32),
            pltpu.VMEM((nbuf, _CH, D), jnp.float32),
            pltpu.VMEM((zrows, D), jnp.float32),
        ] + [pltpu.SemaphoreType.DMA] * nbuf,
    )
    return f(ei3, g)


_BLK = 1280
_NB = _NPAD // _BLK


def _tc_a(degf, emb):
    def body(d0, d1, em, g1_ref):
        d = d0[...] + d1[...] + 1.0
        g1_ref[...] = lax.rsqrt(d) * em[...]

    return pl.pallas_call(
        body,
        grid=(_NB,),
        in_specs=[pl.BlockSpec((_BLK, 16), lambda i: (i, 0)),
                  pl.BlockSpec((_BLK, 16), lambda i: (_NB + i, 0)),
                  pl.BlockSpec((_BLK, 16), lambda i: (i, 0))],
        out_specs=pl.BlockSpec((_BLK, 16), lambda i: (i, 0)),
        out_shape=jax.ShapeDtypeStruct((_NPAD, 16), jnp.float32),
    )(degf, degf, emb)


def _tc_b(degf, scat1f, g1, W1p, b1):
    def body(d0, d1, s0, s1, g1r, w, b, g2_ref):
        dinv16 = lax.rsqrt(d0[...] + d1[...] + 1.0)
        agg = dinv16 * (s0[...] + s1[...] + g1r[...])
        h1 = jnp.maximum(
            jnp.dot(agg, w[...], preferred_element_type=jnp.float32) + b[...],
            0.0)
        g2_ref[...] = dinv16[:, :1] * h1

    return pl.pallas_call(
        body,
        grid=(_NB,),
        in_specs=[pl.BlockSpec((_BLK, 16), lambda i: (i, 0)),
                  pl.BlockSpec((_BLK, 16), lambda i: (_NB + i, 0)),
                  pl.BlockSpec((_BLK, 16), lambda i: (i, 0)),
                  pl.BlockSpec((_BLK, 16), lambda i: (_NB + i, 0)),
                  pl.BlockSpec((_BLK, 16), lambda i: (i, 0)),
                  pl.BlockSpec((16, _H), lambda i: (0, 0)),
                  pl.BlockSpec((1, _H), lambda i: (0, 0))],
        out_specs=pl.BlockSpec((_BLK, _H), lambda i: (i, 0)),
        out_shape=jax.ShapeDtypeStruct((_NPAD, _H), jnp.float32),
    )(degf, degf, scat1f, scat1f, g1, W1p, b1)


def _tc_c(degf, scat2f, g2, W2, b2, W3r, b3):
    def body(d0, d1, s0, s1, g2r, w2, b2r, w3, b3r, out_ref):
        dinv = lax.rsqrt(d0[...] + d1[...] + 1.0)[:, :1]
        agg = dinv * (s0[...] + s1[...] + g2r[...])
        h2 = jnp.maximum(
            jnp.dot(agg, w2[...], preferred_element_type=jnp.float32)
            + b2r[...], 0.0)
        z = jnp.sum(h2 * w3[...], axis=1, keepdims=True) + b3r[...]
        out_ref[...] = jax.nn.sigmoid(z)

    return pl.pallas_call(
        body,
        grid=(_NB,),
        in_specs=[pl.BlockSpec((_BLK, 16), lambda i: (i, 0)),
                  pl.BlockSpec((_BLK, 16), lambda i: (_NB + i, 0)),
                  pl.BlockSpec((_BLK, _H), lambda i: (i, 0)),
                  pl.BlockSpec((_BLK, _H), lambda i: (_NB + i, 0)),
                  pl.BlockSpec((_BLK, _H), lambda i: (i, 0)),
                  pl.BlockSpec((_H, _H), lambda i: (0, 0)),
                  pl.BlockSpec((1, _H), lambda i: (0, 0)),
                  pl.BlockSpec((1, _H), lambda i: (0, 0)),
                  pl.BlockSpec((1, 1), lambda i: (0, 0))],
        out_specs=pl.BlockSpec((_BLK, 1), lambda i: (i, 0)),
        out_shape=jax.ShapeDtypeStruct((_NPAD, 1), jnp.float32),
    )(degf, degf, scat2f, scat2f, g2, W2, b2, W3r, b3)


def kernel(x, edge_index, batch, item_embedding, W1, b1, W2, b2, W3, b3):
    xp = jnp.pad(x[:, 0], (0, _NPAD - _N))
    ei3 = edge_index.reshape(2, _NW * _NCHK, _CH)
    table = jnp.pad(item_embedding, ((0, 0), (0, 16 - _EMBED)))
    W1p = jnp.pad(W1, ((0, 16 - _EMBED), (0, 0)))

    degf, emb = _sc_deg_emb(xp, ei3, table)
    g1 = _tc_a(degf, emb)
    scat1f = _sc_scatter(ei3, g1, 16, 4, _TPW)
    g2 = _tc_b(degf, scat1f, g1, W1p, b1.reshape(1, _H))
    scat2f = _sc_scatter(ei3, g2, _H, 2, 16)
    out = _tc_c(degf, scat2f, g2, W2, b2.reshape(1, _H),
                W3.reshape(1, _H), b3.reshape(1, 1))
    return out[:_N, 0]


# speedup vs baseline: 34.1406x; 1.0002x over previous
import jax
import jax.numpy as jnp
from jax import lax
from jax.experimental import pallas as pl
from jax.experimental.pallas import tpu as pltpu
from jax.experimental.pallas import tpu_sc as plsc

_N = 10000
_VOCAB = 100
_EMBED = 10
_H = 128
_E = 320000

_NW = 32
_NPAD = 10240
_NPW = _NPAD // _NW
_GCH = 80
_CH = 40
_NCHK = _E // (_NW * _CH)
_TPW = _NPAD // 16


def _mesh():
    return plsc.VectorSubcoreMesh(core_axis_name="c", subcore_axis_name="s")


def _ring(nbuf, gather_start, gather_wait, scatter):
    for b in range(nbuf):
        gather_start(b, b)
    fg = (_NCHK - nbuf) // nbuf

    def step(grp, carry):
        for b in range(nbuf):
            t = nbuf * grp + b
            gather_wait(t, b)
            scatter(t, b)
            gather_start(t + nbuf, b)
        return carry

    lax.fori_loop(0, fg, step, 0)
    for t in range(fg * nbuf, _NCHK):
        b = t % nbuf
        gather_wait(t, b)
        scatter(t, b)
        if t + nbuf < _NCHK:
            gather_start(t + nbuf, b)


def _sc_deg_emb(xp, ei3, table):
    def body(x_hbm, ei_hbm, table_hbm, deg_out, emb_out,
             acc, xidx, grows, didx_all, ones, zbuf, sem):
        c = lax.axis_index("c")
        s = lax.axis_index("s")
        wid = s * 2 + c

        pltpu.sync_copy(ei_hbm.at[1, pl.ds(wid * _NCHK, _NCHK)], didx_all)

        def fill_ones(i, carry):
            ones[i] = jnp.ones((16,), jnp.float32)
            return carry

        lax.fori_loop(0, _CH, fill_ones, 0)

        def fill_zero(i, carry):
            zbuf[i] = jnp.zeros((16,), jnp.float32)
            return carry

        lax.fori_loop(0, _TPW, fill_zero, 0)
        pltpu.sync_copy(zbuf, acc.at[pl.ds(s * _TPW, _TPW)])

        for j in range(_NPW // _GCH):
            b = wid * _NPW + j * _GCH
            pltpu.sync_copy(x_hbm.at[pl.ds(b, _GCH)], xidx)
            pltpu.async_copy(table_hbm.at[xidx], grows, sem).wait()
            pltpu.sync_copy(grows, emb_out.at[pl.ds(b, _GCH)])

        plsc.subcore_barrier()

        def deg_step(t, carry):
            pltpu.sync_copy(ones, acc.at[didx_all.at[t]], add=True)
            return carry

        lax.fori_loop(0, _NCHK, deg_step, 0)
        plsc.subcore_barrier()
        pltpu.sync_copy(acc.at[pl.ds(s * _TPW, _TPW)],
                        deg_out.at[pl.ds(c * _NPAD + s * _TPW, _TPW)])

    f = pl.kernel(
        body,
        out_type=[jax.ShapeDtypeStruct((2 * _NPAD, 16), jnp.float32),
                  jax.ShapeDtypeStruct((_NPAD, 16), jnp.float32)],
        mesh=_mesh(),
        compiler_params=pltpu.CompilerParams(use_tc_tiling_on_sc=False),
        scratch_types=[
            pltpu.VMEM_SHARED((_NPAD, 16), jnp.float32),
            pltpu.VMEM((_GCH,), jnp.int32),
            pltpu.VMEM((_GCH, 16), jnp.float32),
            pltpu.VMEM((_NCHK, _CH), jnp.int32),
            pltpu.VMEM((_CH, 16), jnp.float32),
            pltpu.VMEM((_TPW, 16), jnp.float32),
            pltpu.SemaphoreType.DMA,
        ],
    )
    return f(xp, ei3, table)


def _sc_scatter(ei3, g, D, nbuf, zrows):
    nz = _TPW // zrows

    def body(ei_hbm, g_hbm, out, acc, sidx_all, didx_all, rows, zbuf, *sems):
        c = lax.axis_index("c")
        s = lax.axis_index("s")
        wid = s * 2 + c

        pltpu.sync_copy(ei_hbm.at[0, pl.ds(wid * _NCHK, _NCHK)], sidx_all)
        pltpu.sync_copy(ei_hbm.at[1, pl.ds(wid * _NCHK, _NCHK)], didx_all)

        def fill_zero(i, carry):
            for j in range(D // 16):
                zbuf[i, pl.ds(j * 16, 16)] = jnp.zeros((16,), jnp.float32)
            return carry

        lax.fori_loop(0, zrows, fill_zero, 0)
        for k in range(nz):
            pltpu.sync_copy(zbuf, acc.at[pl.ds(s * _TPW + k * zrows, zrows)])
        plsc.subcore_barrier()

        def gather_start(t, b):
            pltpu.make_async_copy(
                g_hbm.at[sidx_all.at[t]], rows.at[b], sems[b]).start()

        def gather_wait(t, b):
            pltpu.make_async_copy(
                g_hbm.at[sidx_all.at[t]], rows.at[b], sems[b]).wait()

        def scatter(t, b):
            pltpu.sync_copy(rows.at[b], acc.at[didx_all.at[t]], add=True)

        _ring(nbuf, gather_start, gather_wait, scatter)

        plsc.subcore_barrier()
        pltpu.sync_copy(acc.at[pl.ds(s * _TPW, _TPW)],
                        out.at[pl.ds(c * _NPAD + s * _TPW, _TPW)])

    f = pl.kernel(
        body,
        out_type=jax.ShapeDtypeStruct((2 * _NPAD, D), jnp.float32),
        mesh=_mesh(),
        compiler_params=pltpu.CompilerParams(use_tc_tiling_on_sc=False),
        scratch_types=[
            pltpu.VMEM_SHARED((_NPAD, D), jnp.float32),
            pltpu.VMEM((_NCHK, _CH), jnp.int32),
            pltpu.VMEM((_NCHK, _CH), jnp.int32),
            pltpu.VMEM((nbuf, _CH, D), jnp.float32),
            pltpu.VMEM((zrows, D), jnp.float32),
        ] + [pltpu.SemaphoreType.DMA] * nbuf,
    )
    return f(ei3, g)


_BLK = 1280
_NB = _NPAD // _BLK


def _tc_b(degf, scat1f, g1, W1p, b1):
    def body(d0, d1, s0, s1, g1r, w, b, g2_ref):
        dinv16 = lax.rsqrt(d0[...] + d1[...] + 1.0)
        agg = dinv16 * (s0[...] + s1[...] + g1r[...])
        h1 = jnp.maximum(
            jnp.dot(agg, w[...], preferred_element_type=jnp.float32) + b[...],
            0.0)
        g2_ref[...] = dinv16[:, :1] * h1

    return pl.pallas_call(
        body,
        grid=(_NB,),
        in_specs=[pl.BlockSpec((_BLK, 16), lambda i: (i, 0)),
                  pl.BlockSpec((_BLK, 16), lambda i: (_NB + i, 0)),
                  pl.BlockSpec((_BLK, 16), lambda i: (i, 0)),
                  pl.BlockSpec((_BLK, 16), lambda i: (_NB + i, 0)),
                  pl.BlockSpec((_BLK, 16), lambda i: (i, 0)),
                  pl.BlockSpec((16, _H), lambda i: (0, 0)),
                  pl.BlockSpec((1, _H), lambda i: (0, 0))],
        out_specs=pl.BlockSpec((_BLK, _H), lambda i: (i, 0)),
        out_shape=jax.ShapeDtypeStruct((_NPAD, _H), jnp.float32),
    )(degf, degf, scat1f, scat1f, g1, W1p, b1)


def _tc_c(degf, scat2f, g2, W2, b2, W3r, b3):
    def body(d0, d1, s0, s1, g2r, w2, b2r, w3, b3r, out_ref):
        dinv = lax.rsqrt(d0[...] + d1[...] + 1.0)[:, :1]
        agg = dinv * (s0[...] + s1[...] + g2r[...])
        h2 = jnp.maximum(
            jnp.dot(agg, w2[...], preferred_element_type=jnp.float32)
            + b2r[...], 0.0)
        z = jnp.sum(h2 * w3[...], axis=1, keepdims=True) + b3r[...]
        out_ref[...] = jax.nn.sigmoid(z)

    return pl.pallas_call(
        body,
        grid=(_NB,),
        in_specs=[pl.BlockSpec((_BLK, 16), lambda i: (i, 0)),
                  pl.BlockSpec((_BLK, 16), lambda i: (_NB + i, 0)),
                  pl.BlockSpec((_BLK, _H), lambda i: (i, 0)),
                  pl.BlockSpec((_BLK, _H), lambda i: (_NB + i, 0)),
                  pl.BlockSpec((_BLK, _H), lambda i: (i, 0)),
                  pl.BlockSpec((_H, _H), lambda i: (0, 0)),
                  pl.BlockSpec((1, _H), lambda i: (0, 0)),
                  pl.BlockSpec((1, _H), lambda i: (0, 0)),
                  pl.BlockSpec((1, 1), lambda i: (0, 0))],
        out_specs=pl.BlockSpec((_BLK, 1), lambda i: (i, 0)),
        out_shape=jax.ShapeDtypeStruct((_NPAD, 1), jnp.float32),
    )(degf, degf, scat2f, scat2f, g2, W2, b2, W3r, b3)


def kernel(x, edge_index, batch, item_embedding, W1, b1, W2, b2, W3, b3):
    xp = jnp.pad(x[:, 0], (0, _NPAD - _N))
    ei3 = edge_index.reshape(2, _NW * _NCHK, _CH)
    table = jnp.pad(item_embedding, ((0, 0), (0, 16 - _EMBED)))
    W1p = jnp.pad(W1, ((0, 16 - _EMBED), (0, 0)))

    degf, emb = _sc_deg_emb(xp, ei3, table)
    g1 = lax.rsqrt(degf[:_NPAD] + degf[_NPAD:] + 1.0) * emb
    scat1f = _sc_scatter(ei3, g1, 16, 4, _TPW)
    g2 = _tc_b(degf, scat1f, g1, W1p, b1.reshape(1, _H))
    scat2f = _sc_scatter(ei3, g2, _H, 4, 16)
    out = _tc_c(degf, scat2f, g2, W2, b2.reshape(1, _H),
                W3.reshape(1, _H), b3.reshape(1, 1))
    return out[:_N, 0]


# speedup vs baseline: 35.0838x; 1.0276x over previous
import jax
import jax.numpy as jnp
from jax import lax
from jax.experimental import pallas as pl
from jax.experimental.pallas import tpu as pltpu
from jax.experimental.pallas import tpu_sc as plsc

_N = 10000
_VOCAB = 100
_EMBED = 10
_H = 128
_E = 320000

_NW = 32
_NPAD = 10240
_NPW = _NPAD // _NW
_GCH = 80
_EPW = _E // _NW
_TPW = _NPAD // 16


def _mesh():
    return plsc.VectorSubcoreMesh(core_axis_name="c", subcore_axis_name="s")


def _ring(nchk, nbuf, gather_start, gather_wait, scatter):
    for b in range(nbuf):
        gather_start(b, b)
    fg = (nchk - nbuf) // nbuf

    def step(grp, carry):
        for b in range(nbuf):
            t = nbuf * grp + b
            gather_wait(t, b)
            scatter(t, b)
            gather_start(t + nbuf, b)
        return carry

    lax.fori_loop(0, fg, step, 0)
    for t in range(fg * nbuf, nchk):
        b = t % nbuf
        gather_wait(t, b)
        scatter(t, b)
        if t + nbuf < nchk:
            gather_start(t + nbuf, b)


_DCH = 80
_DNCHK = _EPW // _DCH


def _sc_deg_emb(xp, ei3, table):
    def body(x_hbm, ei_hbm, table_hbm, deg_out, emb_out,
             acc, xidx, grows, didx_all, ones, zbuf, sem):
        c = lax.axis_index("c")
        s = lax.axis_index("s")
        wid = s * 2 + c

        pltpu.sync_copy(ei_hbm.at[1, pl.ds(wid * _DNCHK, _DNCHK)], didx_all)

        def fill_ones(i, carry):
            ones[i] = jnp.ones((16,), jnp.float32)
            return carry

        lax.fori_loop(0, _DCH, fill_ones, 0)

        def fill_zero(i, carry):
            zbuf[i] = jnp.zeros((16,), jnp.float32)
            return carry

        lax.fori_loop(0, _TPW, fill_zero, 0)
        pltpu.sync_copy(zbuf, acc.at[pl.ds(s * _TPW, _TPW)])

        for j in range(_NPW // _GCH):
            b = wid * _NPW + j * _GCH
            pltpu.sync_copy(x_hbm.at[pl.ds(b, _GCH)], xidx)
            pltpu.async_copy(table_hbm.at[xidx], grows, sem).wait()
            pltpu.sync_copy(grows, emb_out.at[pl.ds(b, _GCH)])

        plsc.subcore_barrier()

        def deg_step(t, carry):
            pltpu.sync_copy(ones, acc.at[didx_all.at[t]], add=True)
            return carry

        lax.fori_loop(0, _DNCHK, deg_step, 0)
        plsc.subcore_barrier()
        pltpu.sync_copy(acc.at[pl.ds(s * _TPW, _TPW)],
                        deg_out.at[pl.ds(c * _NPAD + s * _TPW, _TPW)])

    f = pl.kernel(
        body,
        out_type=[jax.ShapeDtypeStruct((2 * _NPAD, 16), jnp.float32),
                  jax.ShapeDtypeStruct((_NPAD, 16), jnp.float32)],
        mesh=_mesh(),
        compiler_params=pltpu.CompilerParams(use_tc_tiling_on_sc=False),
        scratch_types=[
            pltpu.VMEM_SHARED((_NPAD, 16), jnp.float32),
            pltpu.VMEM((_GCH,), jnp.int32),
            pltpu.VMEM((_GCH, 16), jnp.float32),
            pltpu.VMEM((_DNCHK, _DCH), jnp.int32),
            pltpu.VMEM((_DCH, 16), jnp.float32),
            pltpu.VMEM((_TPW, 16), jnp.float32),
            pltpu.SemaphoreType.DMA,
        ],
    )
    return f(xp, ei3, table)


def _sc_scatter(ei3, g, D, chunk, nbuf, zrows):
    nz = _TPW // zrows
    nchk = _EPW // chunk

    def body(ei_hbm, g_hbm, out, acc, sidx_all, didx_all, rows, zbuf, *sems):
        c = lax.axis_index("c")
        s = lax.axis_index("s")
        wid = s * 2 + c

        pltpu.sync_copy(ei_hbm.at[0, pl.ds(wid * nchk, nchk)], sidx_all)
        pltpu.sync_copy(ei_hbm.at[1, pl.ds(wid * nchk, nchk)], didx_all)

        def fill_zero(i, carry):
            for j in range(D // 16):
                zbuf[i, pl.ds(j * 16, 16)] = jnp.zeros((16,), jnp.float32)
            return carry

        lax.fori_loop(0, zrows, fill_zero, 0)
        for k in range(nz):
            pltpu.sync_copy(zbuf, acc.at[pl.ds(s * _TPW + k * zrows, zrows)])
        plsc.subcore_barrier()

        def gather_start(t, b):
            pltpu.make_async_copy(
                g_hbm.at[sidx_all.at[t]], rows.at[b], sems[b]).start()

        def gather_wait(t, b):
            pltpu.make_async_copy(
                g_hbm.at[sidx_all.at[t]], rows.at[b], sems[b]).wait()

        def scatter(t, b):
            pltpu.sync_copy(rows.at[b], acc.at[didx_all.at[t]], add=True)

        _ring(nchk, nbuf, gather_start, gather_wait, scatter)

        plsc.subcore_barrier()
        pltpu.sync_copy(acc.at[pl.ds(s * _TPW, _TPW)],
                        out.at[pl.ds(c * _NPAD + s * _TPW, _TPW)])

    f = pl.kernel(
        body,
        out_type=jax.ShapeDtypeStruct((2 * _NPAD, D), jnp.float32),
        mesh=_mesh(),
        compiler_params=pltpu.CompilerParams(use_tc_tiling_on_sc=False),
        scratch_types=[
            pltpu.VMEM_SHARED((_NPAD, D), jnp.float32),
            pltpu.VMEM((nchk, chunk), jnp.int32),
            pltpu.VMEM((nchk, chunk), jnp.int32),
            pltpu.VMEM((nbuf, chunk, D), jnp.float32),
            pltpu.VMEM((zrows, D), jnp.float32),
        ] + [pltpu.SemaphoreType.DMA] * nbuf,
    )
    return f(ei3, g)


_BLK = 1280
_NB = _NPAD // _BLK


def _tc_b(agg1, W1p, b1):
    def body(a, w, b, h1_ref):
        h1_ref[...] = jnp.maximum(
            jnp.dot(a[...], w[...], preferred_element_type=jnp.float32)
            + b[...], 0.0)

    return pl.pallas_call(
        body,
        grid=(_NB,),
        in_specs=[pl.BlockSpec((_BLK, 16), lambda i: (i, 0)),
                  pl.BlockSpec((16, _H), lambda i: (0, 0)),
                  pl.BlockSpec((1, _H), lambda i: (0, 0))],
        out_specs=pl.BlockSpec((_BLK, _H), lambda i: (i, 0)),
        out_shape=jax.ShapeDtypeStruct((_NPAD, _H), jnp.float32),
    )(agg1, W1p, b1)


_BLKC = 2000


def _tc_c(agg2, W2, b2, W3r, b3):
    def body(a, w2, b2r, w3, b3r, out_ref):
        h2 = jnp.maximum(
            jnp.dot(a[...], w2[...], preferred_element_type=jnp.float32)
            + b2r[...], 0.0)
        z = jnp.sum(h2 * w3[...], axis=1, keepdims=True) + b3r[...]
        out_ref[...] = jax.nn.sigmoid(z)

    return pl.pallas_call(
        body,
        grid=(_N // _BLKC,),
        in_specs=[pl.BlockSpec((_BLKC, _H), lambda i: (i, 0)),
                  pl.BlockSpec((_H, _H), lambda i: (0, 0)),
                  pl.BlockSpec((1, _H), lambda i: (0, 0)),
                  pl.BlockSpec((1, _H), lambda i: (0, 0)),
                  pl.BlockSpec((1, 1), lambda i: (0, 0))],
        out_specs=pl.BlockSpec((_BLKC, 1), lambda i: (i, 0)),
        out_shape=jax.ShapeDtypeStruct((_N, 1), jnp.float32),
    )(agg2, W2, b2, W3r, b3)


def kernel(x, edge_index, batch, item_embedding, W1, b1, W2, b2, W3, b3):
    xp = jnp.pad(x[:, 0], (0, _NPAD - _N))
    ei80 = edge_index.reshape(2, _NW * _DNCHK, _DCH)
    ei40 = edge_index.reshape(2, _NW * (_EPW // 40), 40)
    table = jnp.pad(item_embedding, ((0, 0), (0, 16 - _EMBED)))
    W1p = jnp.pad(W1, ((0, 16 - _EMBED), (0, 0)))

    degf, emb = _sc_deg_emb(xp, ei80, table)
    dinv16 = lax.rsqrt(degf[:_NPAD] + degf[_NPAD:] + 1.0)
    g1 = dinv16 * emb
    scat1f = _sc_scatter(ei80, g1, 16, 80, 4, _TPW)
    agg1 = dinv16 * (scat1f[:_NPAD] + scat1f[_NPAD:] + g1)
    h1 = _tc_b(agg1, W1p, b1.reshape(1, _H))
    g2 = dinv16[:, :1] * h1
    scat2f = _sc_scatter(ei40, g2, _H, 40, 4, 16)
    agg2 = dinv16[:, :1] * (scat2f[:_NPAD] + scat2f[_NPAD:] + g2)
    out = _tc_c(agg2, W2, b2.reshape(1, _H), W3.reshape(1, _H),
                b3.reshape(1, 1))
    return out[:, 0]


# speedup vs baseline: 37.1273x; 1.0582x over previous
import jax
import jax.numpy as jnp
from jax import lax
from jax.experimental import pallas as pl
from jax.experimental.pallas import tpu as pltpu
from jax.experimental.pallas import tpu_sc as plsc

_N = 10000
_VOCAB = 100
_EMBED = 10
_H = 128
_E = 320000

_NW = 32
_NPAD = 10240
_NPW = _NPAD // _NW
_GCH = 80
_EPW = _E // _NW
_TPW = _NPAD // 16


def _mesh():
    return plsc.VectorSubcoreMesh(core_axis_name="c", subcore_axis_name="s")


def _ring(nchk, nbuf, gather_start, gather_wait, scatter):
    for b in range(nbuf):
        gather_start(b, b)
    fg = (nchk - nbuf) // nbuf

    def step(grp, carry):
        for b in range(nbuf):
            t = nbuf * grp + b
            gather_wait(t, b)
            scatter(t, b)
            gather_start(t + nbuf, b)
        return carry

    lax.fori_loop(0, fg, step, 0)
    for t in range(fg * nbuf, nchk):
        b = t % nbuf
        gather_wait(t, b)
        scatter(t, b)
        if t + nbuf < nchk:
            gather_start(t + nbuf, b)


_DCH = 80
_DNCHK = _EPW // _DCH


def _sc_deg_emb(xp, ei3, table):
    def body(x_hbm, ei_hbm, table_hbm, deg_out, emb_out,
             acc, xidx, grows, didx_all, ones, zbuf, sem):
        c = lax.axis_index("c")
        s = lax.axis_index("s")
        wid = s * 2 + c

        pltpu.sync_copy(ei_hbm.at[1, pl.ds(wid * _DNCHK, _DNCHK)], didx_all)

        def fill_ones(i, carry):
            ones[i] = jnp.ones((16,), jnp.float32)
            return carry

        lax.fori_loop(0, _DCH, fill_ones, 0)

        def fill_zero(i, carry):
            zbuf[i] = jnp.zeros((16,), jnp.float32)
            return carry

        lax.fori_loop(0, _TPW, fill_zero, 0)
        pltpu.sync_copy(zbuf, acc.at[pl.ds(s * _TPW, _TPW)])

        for j in range(_NPW // _GCH):
            b = wid * _NPW + j * _GCH
            pltpu.sync_copy(x_hbm.at[pl.ds(b, _GCH)], xidx)
            pltpu.async_copy(table_hbm.at[xidx], grows, sem).wait()
            pltpu.sync_copy(grows, emb_out.at[pl.ds(b, _GCH)])

        plsc.subcore_barrier()

        def deg_step(t, carry):
            pltpu.sync_copy(ones, acc.at[didx_all.at[t]], add=True)
            return carry

        lax.fori_loop(0, _DNCHK, deg_step, 0)
        plsc.subcore_barrier()
        pltpu.sync_copy(acc.at[pl.ds(s * _TPW, _TPW)],
                        deg_out.at[pl.ds(c * _NPAD + s * _TPW, _TPW)])

    f = pl.kernel(
        body,
        out_type=[jax.ShapeDtypeStruct((2 * _NPAD, 16), jnp.float32),
                  jax.ShapeDtypeStruct((_NPAD, 16), jnp.float32)],
        mesh=_mesh(),
        compiler_params=pltpu.CompilerParams(use_tc_tiling_on_sc=False),
        scratch_types=[
            pltpu.VMEM_SHARED((_NPAD, 16), jnp.float32),
            pltpu.VMEM((_GCH,), jnp.int32),
            pltpu.VMEM((_GCH, 16), jnp.float32),
            pltpu.VMEM((_DNCHK, _DCH), jnp.int32),
            pltpu.VMEM((_DCH, 16), jnp.float32),
            pltpu.VMEM((_TPW, 16), jnp.float32),
            pltpu.SemaphoreType.DMA,
        ],
    )
    return f(xp, ei3, table)


def _sc_scatter(ei3, g, D, chunk, nbuf, zrows):
    nz = _TPW // zrows
    nchk = _EPW // chunk

    def body(ei_hbm, g_hbm, out, acc, sidx_all, didx_all, rows, zbuf, *sems):
        c = lax.axis_index("c")
        s = lax.axis_index("s")
        wid = s * 2 + c

        pltpu.sync_copy(ei_hbm.at[0, pl.ds(wid * nchk, nchk)], sidx_all)
        pltpu.sync_copy(ei_hbm.at[1, pl.ds(wid * nchk, nchk)], didx_all)

        def fill_zero(i, carry):
            for j in range(D // 16):
                zbuf[i, pl.ds(j * 16, 16)] = jnp.zeros((16,), jnp.float32)
            return carry

        lax.fori_loop(0, zrows, fill_zero, 0)
        for k in range(nz):
            pltpu.sync_copy(zbuf, acc.at[pl.ds(s * _TPW + k * zrows, zrows)])
        plsc.subcore_barrier()

        def gather_start(t, b):
            pltpu.make_async_copy(
                g_hbm.at[sidx_all.at[t]], rows.at[b], sems[b]).start()

        def gather_wait(t, b):
            pltpu.make_async_copy(
                g_hbm.at[sidx_all.at[t]], rows.at[b], sems[b]).wait()

        def scatter(t, b):
            pltpu.sync_copy(rows.at[b], acc.at[didx_all.at[t]], add=True)

        _ring(nchk, nbuf, gather_start, gather_wait, scatter)

        plsc.subcore_barrier()
        pltpu.sync_copy(acc.at[pl.ds(s * _TPW, _TPW)],
                        out.at[pl.ds(c * _NPAD + s * _TPW, _TPW)])

    f = pl.kernel(
        body,
        out_type=jax.ShapeDtypeStruct((2 * _NPAD, D), jnp.float32),
        mesh=_mesh(),
        compiler_params=pltpu.CompilerParams(use_tc_tiling_on_sc=False),
        scratch_types=[
            pltpu.VMEM_SHARED((_NPAD, D), jnp.float32),
            pltpu.VMEM((nchk, chunk), jnp.int32),
            pltpu.VMEM((nchk, chunk), jnp.int32),
            pltpu.VMEM((nbuf, chunk, D), jnp.float32),
            pltpu.VMEM((zrows, D), jnp.float32),
        ] + [pltpu.SemaphoreType.DMA] * nbuf,
    )
    return f(ei3, g)


_BLK = 1280
_NB = _NPAD // _BLK


def _tc_b(agg1, W1p, b1):
    def body(a, w, b, h1_ref):
        h1_ref[...] = jnp.maximum(
            jnp.dot(a[...], w[...], preferred_element_type=jnp.float32)
            + b[...], 0.0)

    return pl.pallas_call(
        body,
        grid=(_NB,),
        in_specs=[pl.BlockSpec((_BLK, 16), lambda i: (i, 0)),
                  pl.BlockSpec((16, _H), lambda i: (0, 0)),
                  pl.BlockSpec((1, _H), lambda i: (0, 0))],
        out_specs=pl.BlockSpec((_BLK, _H), lambda i: (i, 0)),
        out_shape=jax.ShapeDtypeStruct((_NPAD, _H), jnp.float32),
    )(agg1, W1p, b1)


_BLKC = 2000


def _tc_c(agg2, W2, b2, W3r, b3):
    def body(a, w2, b2r, w3, b3r, out_ref):
        h2 = jnp.maximum(
            jnp.dot(a[...], w2[...], preferred_element_type=jnp.float32)
            + b2r[...], 0.0)
        z = jnp.sum(h2 * w3[...], axis=1, keepdims=True) + b3r[...]
        out_ref[...] = jax.nn.sigmoid(z)

    return pl.pallas_call(
        body,
        grid=(_N // _BLKC,),
        in_specs=[pl.BlockSpec((_BLKC, _H), lambda i: (i, 0)),
                  pl.BlockSpec((_H, _H), lambda i: (0, 0)),
                  pl.BlockSpec((1, _H), lambda i: (0, 0)),
                  pl.BlockSpec((1, _H), lambda i: (0, 0)),
                  pl.BlockSpec((1, 1), lambda i: (0, 0))],
        out_specs=pl.BlockSpec((_BLKC, 1), lambda i: (i, 0)),
        out_shape=jax.ShapeDtypeStruct((_N, 1), jnp.float32),
    )(agg2, W2, b2, W3r, b3)


def kernel(x, edge_index, batch, item_embedding, W1, b1, W2, b2, W3, b3):
    xp = jnp.pad(x[:, 0], (0, _NPAD - _N))
    ei80 = edge_index.reshape(2, _NW * _DNCHK, _DCH)
    ei40 = edge_index.reshape(2, _NW * (_EPW // 40), 40)
    table = jnp.pad(item_embedding, ((0, 0), (0, 16 - _EMBED)))
    W1p = jnp.pad(W1, ((0, 16 - _EMBED), (0, 0)))

    degf, emb = _sc_deg_emb(xp, ei80, table)
    dinv16 = lax.rsqrt(degf[:_NPAD] + degf[_NPAD:] + 1.0)
    g1 = dinv16 * emb
    scat1f = _sc_scatter(ei80, g1, 16, 80, 6, _TPW)
    agg1 = dinv16 * (scat1f[:_NPAD] + scat1f[_NPAD:] + g1)
    h1 = _tc_b(agg1, W1p, b1.reshape(1, _H))
    g2 = dinv16[:, :1] * h1
    scat2f = _sc_scatter(ei40, g2, _H, 40, 5, 16)
    agg2 = dinv16[:, :1] * (scat2f[:_NPAD] + scat2f[_NPAD:] + g2)
    return _tc_c(agg2, W2, b2.reshape(1, _H), W3.reshape(1, _H),
                 b3.reshape(1, 1))[:, 0]


# speedup vs baseline: 39.1087x; 1.0534x over previous
import jax
import jax.numpy as jnp
from jax import lax
from jax.experimental import pallas as pl
from jax.experimental.pallas import tpu as pltpu
from jax.experimental.pallas import tpu_sc as plsc

_N = 10000
_VOCAB = 100
_EMBED = 10
_H = 128
_E = 320000

_NW = 32
_NPAD = 10240
_NPW = _NPAD // _NW
_GCH = 80
_EPW = _E // _NW
_TPW = _NPAD // 16


def _mesh():
    return plsc.VectorSubcoreMesh(core_axis_name="c", subcore_axis_name="s")


def _ring(nchk, nbuf, gather_start, gather_wait, scatter):
    for b in range(nbuf):
        gather_start(b, b)
    fg = (nchk - nbuf) // nbuf

    def step(grp, carry):
        for b in range(nbuf):
            t = nbuf * grp + b
            gather_wait(t, b)
            scatter(t, b)
            gather_start(t + nbuf, b)
        return carry

    lax.fori_loop(0, fg, step, 0)
    for t in range(fg * nbuf, nchk):
        b = t % nbuf
        gather_wait(t, b)
        scatter(t, b)
        if t + nbuf < nchk:
            gather_start(t + nbuf, b)


_DCH = 80
_DNCHK = 2 * _EPW // _DCH
_NGRP = _NPAD // 16 // 16


def _sc_deg_emb(xp, ei, table):
    epw = 2 * _EPW

    def body(x_hbm, ei_hbm, table_hbm, dinv_out, emb_out,
             partials, degloc, didx_all, pbuf, dinvbuf, xidx, grows, sem):
        c = lax.axis_index("c")
        s = lax.axis_index("s")
        wid = s * 2 + c
        ones16 = jnp.ones((16,), jnp.float32)

        pltpu.sync_copy(ei_hbm.at[1, pl.ds(s * epw, epw)], didx_all)

        def zero_deg(i, carry):
            degloc[pl.ds(i * 16, 16)] = jnp.zeros((16,), jnp.float32)
            return carry

        lax.fori_loop(0, _NPAD // 16, zero_deg, 0)

        def deg_step(t, carry):
            for j in range(4):
                idxv = didx_all[pl.ds(t * 64 + j * 16, 16)]
                plsc.addupdate_scatter(degloc, [idxv], ones16)
            return carry

        lax.fori_loop(0, epw // 64, deg_step, 0)
        pltpu.sync_copy(degloc, partials.at[s])

        for j in range(_NPW // _GCH):
            b = wid * _NPW + j * _GCH
            pltpu.sync_copy(x_hbm.at[pl.ds(b, _GCH)], xidx)
            pltpu.async_copy(table_hbm.at[xidx], grows, sem).wait()
            pltpu.sync_copy(grows, emb_out.at[pl.ds(b, _GCH)])

        plsc.subcore_barrier()

        for p in range(16):
            pltpu.sync_copy(partials.at[p, pl.ds(s * _TPW, _TPW)],
                            pbuf.at[pl.ds(p * _TPW, _TPW)])

        def dinv_step(grp, carry):
            d = pbuf[pl.ds(grp * 16, 16)]
            for p in range(1, 16):
                d = d + pbuf[pl.ds(p * _TPW + grp * 16, 16)]
            d = d + 1.0
            yi = 1597463007 - jnp.right_shift(plsc.bitcast(d, jnp.int32), 1)
            y = plsc.bitcast(yi, jnp.float32)
            h = d * 0.5
            for _ in range(3):
                y = y * (1.5 - h * y * y)
            dinvbuf[pl.ds(grp * 16, 16)] = y
            return carry

        lax.fori_loop(0, _TPW // 16, dinv_step, 0)

        @pl.when(c == 0)
        def _():
            pltpu.sync_copy(dinvbuf, dinv_out.at[pl.ds(s * _TPW, _TPW)])

    f = pl.kernel(
        body,
        out_type=[jax.ShapeDtypeStruct((_NPAD,), jnp.float32),
                  jax.ShapeDtypeStruct((_NPAD, 16), jnp.float32)],
        mesh=_mesh(),
        compiler_params=pltpu.CompilerParams(
            use_tc_tiling_on_sc=False, needs_layout_passes=False),
        scratch_types=[
            pltpu.VMEM_SHARED((16, _NPAD), jnp.float32),
            pltpu.VMEM((_NPAD,), jnp.float32),
            pltpu.VMEM((epw,), jnp.int32),
            pltpu.VMEM((16 * _TPW,), jnp.float32),
            pltpu.VMEM((_TPW,), jnp.float32),
            pltpu.VMEM((_GCH,), jnp.int32),
            pltpu.VMEM((_GCH, 16), jnp.float32),
            pltpu.SemaphoreType.DMA,
        ],
    )
    return f(xp, ei, table)


def _sc_scatter(ei3, g, D, chunk, nbuf, zrows):
    nz = _TPW // zrows
    nchk = _EPW // chunk

    def body(ei_hbm, g_hbm, out, acc, sidx_all, didx_all, rows, zbuf, *sems):
        c = lax.axis_index("c")
        s = lax.axis_index("s")
        wid = s * 2 + c

        pltpu.sync_copy(ei_hbm.at[0, pl.ds(wid * nchk, nchk)], sidx_all)
        pltpu.sync_copy(ei_hbm.at[1, pl.ds(wid * nchk, nchk)], didx_all)

        def fill_zero(i, carry):
            for j in range(D // 16):
                zbuf[i, pl.ds(j * 16, 16)] = jnp.zeros((16,), jnp.float32)
            return carry

        lax.fori_loop(0, zrows, fill_zero, 0)
        for k in range(nz):
            pltpu.sync_copy(zbuf, acc.at[pl.ds(s * _TPW + k * zrows, zrows)])
        plsc.subcore_barrier()

        def gather_start(t, b):
            pltpu.make_async_copy(
                g_hbm.at[sidx_all.at[t]], rows.at[b], sems[b]).start()

        def gather_wait(t, b):
            pltpu.make_async_copy(
                g_hbm.at[sidx_all.at[t]], rows.at[b], sems[b]).wait()

        def scatter(t, b):
            pltpu.sync_copy(rows.at[b], acc.at[didx_all.at[t]], add=True)

        _ring(nchk, nbuf, gather_start, gather_wait, scatter)

        plsc.subcore_barrier()
        pltpu.sync_copy(acc.at[pl.ds(s * _TPW, _TPW)],
                        out.at[pl.ds(c * _NPAD + s * _TPW, _TPW)])

    f = pl.kernel(
        body,
        out_type=jax.ShapeDtypeStruct((2 * _NPAD, D), jnp.float32),
        mesh=_mesh(),
        compiler_params=pltpu.CompilerParams(use_tc_tiling_on_sc=False),
        scratch_types=[
            pltpu.VMEM_SHARED((_NPAD, D), jnp.float32),
            pltpu.VMEM((nchk, chunk), jnp.int32),
            pltpu.VMEM((nchk, chunk), jnp.int32),
            pltpu.VMEM((nbuf, chunk, D), jnp.float32),
            pltpu.VMEM((zrows, D), jnp.float32),
        ] + [pltpu.SemaphoreType.DMA] * nbuf,
    )
    return f(ei3, g)


_BLK = 1280
_NB = _NPAD // _BLK


def _tc_b(agg1, W1p, b1):
    def body(a, w, b, h1_ref):
        h1_ref[...] = jnp.maximum(
            jnp.dot(a[...], w[...], preferred_element_type=jnp.float32)
            + b[...], 0.0)

    return pl.pallas_call(
        body,
        grid=(_NB,),
        in_specs=[pl.BlockSpec((_BLK, 16), lambda i: (i, 0)),
                  pl.BlockSpec((16, _H), lambda i: (0, 0)),
                  pl.BlockSpec((1, _H), lambda i: (0, 0))],
        out_specs=pl.BlockSpec((_BLK, _H), lambda i: (i, 0)),
        out_shape=jax.ShapeDtypeStruct((_NPAD, _H), jnp.float32),
    )(agg1, W1p, b1)


_BLKC = 2000


def _tc_c(agg2, W2, b2, W3r, b3):
    def body(a, w2, b2r, w3, b3r, out_ref):
        h2 = jnp.maximum(
            jnp.dot(a[...], w2[...], preferred_element_type=jnp.float32)
            + b2r[...], 0.0)
        z = jnp.sum(h2 * w3[...], axis=1, keepdims=True) + b3r[...]
        out_ref[...] = jax.nn.sigmoid(z)

    return pl.pallas_call(
        body,
        grid=(_N // _BLKC,),
        in_specs=[pl.BlockSpec((_BLKC, _H), lambda i: (i, 0)),
                  pl.BlockSpec((_H, _H), lambda i: (0, 0)),
                  pl.BlockSpec((1, _H), lambda i: (0, 0)),
                  pl.BlockSpec((1, _H), lambda i: (0, 0)),
                  pl.BlockSpec((1, 1), lambda i: (0, 0))],
        out_specs=pl.BlockSpec((_BLKC, 1), lambda i: (i, 0)),
        out_shape=jax.ShapeDtypeStruct((_N, 1), jnp.float32),
    )(agg2, W2, b2, W3r, b3)


def kernel(x, edge_index, batch, item_embedding, W1, b1, W2, b2, W3, b3):
    xp = jnp.pad(x[:, 0], (0, _NPAD - _N))
    ei80 = edge_index.reshape(2, _E // _DCH, _DCH)
    ei40 = edge_index.reshape(2, _NW * (_EPW // 40), 40)
    table = jnp.pad(item_embedding, ((0, 0), (0, 16 - _EMBED)))
    W1p = jnp.pad(W1, ((0, 16 - _EMBED), (0, 0)))

    dinv, emb = _sc_deg_emb(xp, edge_index, table)
    dinv1 = dinv[:, None]
    g1 = dinv1 * emb
    scat1f = _sc_scatter(ei80, g1, 16, 80, 6, _TPW)
    agg1 = dinv1 * (scat1f[:_NPAD] + scat1f[_NPAD:] + g1)
    h1 = _tc_b(agg1, W1p, b1.reshape(1, _H))
    g2 = dinv1 * h1
    scat2f = _sc_scatter(ei40, g2, _H, 40, 5, 16)
    agg2 = dinv1 * (scat2f[:_NPAD] + scat2f[_NPAD:] + g2)
    return _tc_c(agg2, W2, b2.reshape(1, _H), W3.reshape(1, _H),
                 b3.reshape(1, 1))[:, 0]
